# baseline (speedup 1.0000x reference)
import jax
import jax.numpy as jnp
from jax.experimental import pallas as pl
from jax.experimental.pallas import tpu as pltpu

_B, _S, _E = 64, 1024, 1024
_T = 768
_NE = 16
_K = 2
_NBUF = 2
_CB = 4
_NCH = _B // _CB


def _gate_kernel(x_hbm, text_ref, w1_hbm, b1_ref, w2_ref, b2_ref,
                 w_out_ref, i_out_ref, l_out_ref,
                 buf, w1_v, acc_ref, ht_ref, h_ref, sems, w1_sem):
    for r in range(_NBUF):
        pltpu.make_async_copy(
            x_hbm.at[pl.ds(r * _CB, _CB)], buf.at[r], sems.at[r]).start()
    pltpu.make_async_copy(w1_hbm, w1_v, w1_sem).start()

    def outer(o, carry):
        @pl.when(o == 2)
        def _text_stage():
            pltpu.make_async_copy(w1_hbm, w1_v, w1_sem).wait()
            w1b = w1_v[_E:_E + _T, :]
            ht_ref[...] = (jnp.dot(text_ref[...], w1b,
                                   preferred_element_type=jnp.float32)
                           + b1_ref[...])

        @pl.when(o == 6)
        def _partial_h_stage():
            w1a = w1_v[0:_E, :]
            mean0 = acc_ref[0:48, :] * (1.0 / _S)
            h0 = (jnp.dot(mean0, w1a, preferred_element_type=jnp.float32)
                  + ht_ref[0:48, :])
            h_ref[0:48, :] = jnp.maximum(h0, 0.0)

        for r in range(_NBUF):
            c = o * _NBUF + r
            pltpu.make_async_copy(
                x_hbm.at[pl.ds(c * _CB, _CB)], buf.at[r], sems.at[r]).wait()
            s = jnp.sum(buf[r], axis=1)
            for q in range(_CB):
                acc_ref[pl.ds(c * _CB + q, 1), :] = s[q:q + 1]
            nc = c + _NBUF

            @pl.when(nc < _NCH)
            def _():
                pltpu.make_async_copy(
                    x_hbm.at[pl.ds(nc * _CB, _CB)], buf.at[r], sems.at[r]).start()
        return carry

    jax.lax.fori_loop(0, _NCH // _NBUF, outer, 0)

    w1a = w1_v[0:_E, :]
    mean1 = acc_ref[48:_B, :] * (1.0 / _S)
    h1 = (jnp.dot(mean1, w1a, preferred_element_type=jnp.float32)
          + ht_ref[48:_B, :])
    h_ref[48:_B, :] = jnp.maximum(h1, 0.0)
    logits = (jnp.dot(h_ref[...], w2_ref[...],
                      preferred_element_type=jnp.float32)
              + b2_ref[...])
    l_out_ref[...] = logits

    lane = jax.lax.broadcasted_iota(jnp.int32, (_B, _NE), 1)
    m1 = jnp.max(logits, axis=1, keepdims=True)
    i1 = jnp.min(jnp.where(logits == m1, lane, _NE), axis=1, keepdims=True)
    masked = jnp.where(lane == i1, -jnp.inf, logits)
    m2 = jnp.max(masked, axis=1, keepdims=True)
    i2 = jnp.min(jnp.where(masked == m2, lane, _NE), axis=1, keepdims=True)

    lane2 = jax.lax.broadcasted_iota(jnp.int32, (_B, _K), 1)
    i_out_ref[...] = jnp.where(lane2 == 0, i1, i2)
    e2 = jnp.exp(m2 - m1)
    denom = 1.0 + e2
    w_out_ref[...] = jnp.where(lane2 == 0, 1.0 / denom, e2 / denom)


def kernel(x, text_embedding, W1, b1, W2, b2):
    b1r = b1.reshape(1, _E)
    b2r = b2.reshape(1, _NE)
    out_shape = (
        jax.ShapeDtypeStruct((_B, _K), jnp.float32),
        jax.ShapeDtypeStruct((_B, _K), jnp.int32),
        jax.ShapeDtypeStruct((_B, _NE), jnp.float32),
    )
    weights, indices, logits = pl.pallas_call(
        _gate_kernel,
        in_specs=[
            pl.BlockSpec(memory_space=pl.ANY),
            pl.BlockSpec(memory_space=pltpu.MemorySpace.VMEM),
            pl.BlockSpec(memory_space=pl.ANY),
            pl.BlockSpec(memory_space=pltpu.MemorySpace.VMEM),
            pl.BlockSpec(memory_space=pltpu.MemorySpace.VMEM),
            pl.BlockSpec(memory_space=pltpu.MemorySpace.VMEM),
        ],
        out_specs=(
            pl.BlockSpec(memory_space=pltpu.MemorySpace.VMEM),
            pl.BlockSpec(memory_space=pltpu.MemorySpace.VMEM),
            pl.BlockSpec(memory_space=pltpu.MemorySpace.VMEM),
        ),
        out_shape=out_shape,
        scratch_shapes=[
            pltpu.VMEM((_NBUF, _CB, _S, _E), jnp.float32),
            pltpu.VMEM((_E + _T, _E), jnp.float32),
            pltpu.VMEM((_B, _E), jnp.float32),
            pltpu.VMEM((_B, _E), jnp.float32),
            pltpu.VMEM((_B, _E), jnp.float32),
            pltpu.SemaphoreType.DMA((_NBUF,)),
            pltpu.SemaphoreType.DMA,
        ],
    )(x, text_embedding, W1, b1r, W2, b2r)
    return (weights, indices, logits)


# speedup vs baseline: 1.0004x; 1.0004x over previous
import jax
import jax.numpy as jnp
from jax.experimental import pallas as pl
from jax.experimental.pallas import tpu as pltpu

_B, _S, _E = 64, 1024, 1024
_T = 768
_NE = 16
_K = 2
_NBUF = 4
_CB = 2
_NCH = _B // _CB


def _gate_kernel(x_hbm, text_ref, w1_hbm, b1_ref, w2_ref, b2_ref,
                 w_out_ref, i_out_ref, l_out_ref,
                 buf, w1_v, acc_ref, ht_ref, h_ref, sems, w1_sem):
    for r in range(_NBUF):
        pltpu.make_async_copy(
            x_hbm.at[pl.ds(r * _CB, _CB)], buf.at[r], sems.at[r]).start()
    pltpu.make_async_copy(w1_hbm, w1_v, w1_sem).start()

    def outer(o, carry):
        @pl.when(o == 2)
        def _text_stage():
            pltpu.make_async_copy(w1_hbm, w1_v, w1_sem).wait()
            w1b = w1_v[_E:_E + _T, :]
            ht_ref[...] = (jnp.dot(text_ref[...], w1b,
                                   preferred_element_type=jnp.float32)
                           + b1_ref[...])

        @pl.when(o == 6)
        def _partial_h_stage():
            w1a = w1_v[0:_E, :]
            mean0 = acc_ref[0:48, :] * (1.0 / _S)
            h0 = (jnp.dot(mean0, w1a, preferred_element_type=jnp.float32)
                  + ht_ref[0:48, :])
            h_ref[0:48, :] = jnp.maximum(h0, 0.0)

        for r in range(_NBUF):
            c = o * _NBUF + r
            pltpu.make_async_copy(
                x_hbm.at[pl.ds(c * _CB, _CB)], buf.at[r], sems.at[r]).wait()
            s = jnp.sum(buf[r], axis=1)
            for q in range(_CB):
                acc_ref[pl.ds(c * _CB + q, 1), :] = s[q:q + 1]
            nc = c + _NBUF

            @pl.when(nc < _NCH)
            def _():
                pltpu.make_async_copy(
                    x_hbm.at[pl.ds(nc * _CB, _CB)], buf.at[r], sems.at[r]).start()
        return carry

    jax.lax.fori_loop(0, _NCH // _NBUF, outer, 0)

    w1a = w1_v[0:_E, :]
    mean1 = acc_ref[48:_B, :] * (1.0 / _S)
    h1 = (jnp.dot(mean1, w1a, preferred_element_type=jnp.float32)
          + ht_ref[48:_B, :])
    h_ref[48:_B, :] = jnp.maximum(h1, 0.0)
    logits = (jnp.dot(h_ref[...], w2_ref[...],
                      preferred_element_type=jnp.float32)
              + b2_ref[...])
    l_out_ref[...] = logits

    lane = jax.lax.broadcasted_iota(jnp.int32, (_B, _NE), 1)
    m1 = jnp.max(logits, axis=1, keepdims=True)
    i1 = jnp.min(jnp.where(logits == m1, lane, _NE), axis=1, keepdims=True)
    masked = jnp.where(lane == i1, -jnp.inf, logits)
    m2 = jnp.max(masked, axis=1, keepdims=True)
    i2 = jnp.min(jnp.where(masked == m2, lane, _NE), axis=1, keepdims=True)

    lane2 = jax.lax.broadcasted_iota(jnp.int32, (_B, _K), 1)
    i_out_ref[...] = jnp.where(lane2 == 0, i1, i2)
    e2 = jnp.exp(m2 - m1)
    denom = 1.0 + e2
    w_out_ref[...] = jnp.where(lane2 == 0, 1.0 / denom, e2 / denom)


def kernel(x, text_embedding, W1, b1, W2, b2):
    b1r = b1.reshape(1, _E)
    b2r = b2.reshape(1, _NE)
    out_shape = (
        jax.ShapeDtypeStruct((_B, _K), jnp.float32),
        jax.ShapeDtypeStruct((_B, _K), jnp.int32),
        jax.ShapeDtypeStruct((_B, _NE), jnp.float32),
    )
    weights, indices, logits = pl.pallas_call(
        _gate_kernel,
        in_specs=[
            pl.BlockSpec(memory_space=pl.ANY),
            pl.BlockSpec(memory_space=pltpu.MemorySpace.VMEM),
            pl.BlockSpec(memory_space=pl.ANY),
            pl.BlockSpec(memory_space=pltpu.MemorySpace.VMEM),
            pl.BlockSpec(memory_space=pltpu.MemorySpace.VMEM),
            pl.BlockSpec(memory_space=pltpu.MemorySpace.VMEM),
        ],
        out_specs=(
            pl.BlockSpec(memory_space=pltpu.MemorySpace.VMEM),
            pl.BlockSpec(memory_space=pltpu.MemorySpace.VMEM),
            pl.BlockSpec(memory_space=pltpu.MemorySpace.VMEM),
        ),
        out_shape=out_shape,
        scratch_shapes=[
            pltpu.VMEM((_NBUF, _CB, _S, _E), jnp.float32),
            pltpu.VMEM((_E + _T, _E), jnp.float32),
            pltpu.VMEM((_B, _E), jnp.float32),
            pltpu.VMEM((_B, _E), jnp.float32),
            pltpu.VMEM((_B, _E), jnp.float32),
            pltpu.SemaphoreType.DMA((_NBUF,)),
            pltpu.SemaphoreType.DMA,
        ],
    )(x, text_embedding, W1, b1r, W2, b2r)
    return (weights, indices, logits)
